# Initial kernel scaffold; baseline (speedup 1.0000x reference)
import jax
import jax.numpy as jnp
from jax.experimental import pallas as pl


def kernel(inputs, sampled_edges, msg_fc1_w, msg_fc1_b, msg_fc2_w, msg_fc2_b, hidden_r_w, hidden_i_w, hidden_h_w, input_r_w, input_r_b, input_i_w, input_i_b, input_n_w, input_n_b, out_fc1_w, out_fc1_b, out_fc2_w, out_fc2_b, out_fc3_w, out_fc3_b):
    raise NotImplementedError("write your pallas kernel here")


# speedup vs baseline: 3.7621x; 3.7621x over previous
import numpy as np

import jax
import jax.numpy as jnp
from jax.experimental import pallas as pl
from jax.experimental.pallas import tpu as pltpu


def _decoder_kernel(T, V, Vp, NT,
                    ins0_ref, aggw_ref, wr_ref, ws_ref, b1_ref, f2_ref, b2_ref,
                    hrw_ref, hiw_ref, hhw_ref,
                    irw_ref, irb_ref, iiw_ref, iib_ref, inw_ref, inb_ref,
                    o1w_ref, o1b_ref, o2w_ref, o2b_ref, o3w_ref, o3b_ref,
                    out_ref):
    H = hrw_ref.shape[0]
    f32 = jnp.float32
    h = jnp.zeros((V, H), f32)
    ins = ins0_ref[0]
    zpad = jnp.zeros((Vp - V, H), f32)
    for t in range(T):
        hp = jnp.concatenate([h, zpad], axis=0)
        agg = jnp.zeros((V, H), f32)
        for i in range(NT):
            xr = jnp.dot(hp, wr_ref[i], preferred_element_type=f32) + b1_ref[i]
            xs = jnp.dot(h, ws_ref[i], preferred_element_type=f32)
            pre = xs[:, None, :] + xr[None, :, :]
            a1 = jnp.tanh(pre).reshape(V * Vp, H)
            a2 = jnp.tanh(jnp.dot(a1, f2_ref[i], preferred_element_type=f32)
                          + b2_ref[i])
            agg = agg + jnp.dot(aggw_ref[0, i], a2, preferred_element_type=f32)
        inp_r = jnp.dot(ins, irw_ref[...], preferred_element_type=f32) + irb_ref[...]
        inp_i = jnp.dot(ins, iiw_ref[...], preferred_element_type=f32) + iib_ref[...]
        inp_n = jnp.dot(ins, inw_ref[...], preferred_element_type=f32) + inb_ref[...]
        r = jax.nn.sigmoid(inp_r + jnp.dot(agg, hrw_ref[...], preferred_element_type=f32))
        ig = jax.nn.sigmoid(inp_i + jnp.dot(agg, hiw_ref[...], preferred_element_type=f32))
        n = jnp.tanh(inp_n + r * jnp.dot(agg, hhw_ref[...], preferred_element_type=f32))
        h = (1.0 - ig) * n + ig * h
        p = jax.nn.relu(jnp.dot(h, o1w_ref[...], preferred_element_type=f32) + o1b_ref[...])
        p = jax.nn.relu(jnp.dot(p, o2w_ref[...], preferred_element_type=f32) + o2b_ref[...])
        p = jnp.dot(p, o3w_ref[...], preferred_element_type=f32) + o3b_ref[...]
        ins = ins + p
        out_ref[0, t] = ins


def kernel(inputs, sampled_edges, msg_fc1_w, msg_fc1_b, msg_fc2_w, msg_fc2_b,
           hidden_r_w, hidden_i_w, hidden_h_w, input_r_w, input_r_b,
           input_i_w, input_i_b, input_n_w, input_n_b, out_fc1_w, out_fc1_b,
           out_fc2_w, out_fc2_b, out_fc3_w, out_fc3_b):
    B, T, V, DIN = inputs.shape
    H = hidden_r_w.shape[0]
    ET = msg_fc1_w.shape[0]
    NT = ET - 1
    Vp = ((V + 7) // 8) * 8
    E = V * (V - 1)

    e = np.arange(E)
    s_idx = e // (V - 1)
    k = e % (V - 1)
    r_idx = k + (k >= s_idx)

    scale = 1.0 / ((ET - 1.0) * (V - 1.0))
    vals = jnp.transpose(sampled_edges[:, :, 1:], (0, 2, 1)) * scale
    aggw = jnp.zeros((B, NT, V, V * Vp), dtype=jnp.float32)
    aggw = aggw.at[:, :, r_idx, s_idx * Vp + r_idx].set(vals)

    wr = msg_fc1_w[1:, :H, :]
    ws = msg_fc1_w[1:, H:, :]
    b1 = msg_fc1_b[1:].reshape(NT, 1, H)
    f2 = msg_fc2_w[1:]
    b2 = msg_fc2_b[1:].reshape(NT, 1, H)
    ins0 = inputs[:, 0]

    def vec(x):
        return x.reshape(1, -1)

    def rep(shape):
        nd = len(shape)
        return pl.BlockSpec(shape, lambda b, _n=nd: (0,) * _n)

    import functools
    body = functools.partial(_decoder_kernel, T, V, Vp, NT)

    return pl.pallas_call(
        body,
        grid=(B,),
        in_specs=[
            pl.BlockSpec((1, V, DIN), lambda b: (b, 0, 0)),
            pl.BlockSpec((1, NT, V, V * Vp), lambda b: (b, 0, 0, 0)),
            rep((NT, H, H)), rep((NT, H, H)), rep((NT, 1, H)),
            rep((NT, H, H)), rep((NT, 1, H)),
            rep((H, H)), rep((H, H)), rep((H, H)),
            rep((DIN, H)), rep((1, H)),
            rep((DIN, H)), rep((1, H)),
            rep((DIN, H)), rep((1, H)),
            rep((H, H)), rep((1, H)),
            rep((H, H)), rep((1, H)),
            rep((H, DIN)), rep((1, DIN)),
        ],
        out_specs=pl.BlockSpec((1, T, V, DIN), lambda b: (b, 0, 0, 0)),
        out_shape=jax.ShapeDtypeStruct((B, T, V, DIN), jnp.float32),
        compiler_params=pltpu.CompilerParams(
            dimension_semantics=("parallel",)),
    )(ins0, aggw, wr, ws, b1, f2, b2,
      hidden_r_w, hidden_i_w, hidden_h_w,
      input_r_w, vec(input_r_b), input_i_w, vec(input_i_b),
      input_n_w, vec(input_n_b),
      out_fc1_w, vec(out_fc1_b), out_fc2_w, vec(out_fc2_b),
      out_fc3_w, vec(out_fc3_b))


# speedup vs baseline: 8.0803x; 2.1478x over previous
import numpy as np

import jax
import jax.numpy as jnp
from jax.experimental import pallas as pl
from jax.experimental.pallas import tpu as pltpu


def _decoder_kernel(T, V, Vp, NT,
                    ins0_ref, wt_ref, mask_ref, rmat_ref,
                    wr_ref, ws_ref, b1_ref, f2_ref, b2_ref,
                    hrw_ref, hiw_ref, hhw_ref,
                    irw_ref, irb_ref, iiw_ref, iib_ref, inw_ref, inb_ref,
                    o1w_ref, o1b_ref, o2w_ref, o2b_ref, o3w_ref, o3b_ref,
                    out_ref):
    H = hrw_ref.shape[0]
    f32 = jnp.float32
    aggws = [
        mask_ref[...] * jnp.dot(wt_ref[0, i], rmat_ref[...],
                                preferred_element_type=f32)
        for i in range(NT)
    ]
    h = jnp.zeros((V, H), f32)
    ins = ins0_ref[0]
    zpad = jnp.zeros((Vp - V, H), f32)
    for t in range(T):
        hp = jnp.concatenate([h, zpad], axis=0)
        agg = jnp.zeros((V, H), f32)
        for i in range(NT):
            xr = jnp.dot(hp, wr_ref[i], preferred_element_type=f32) + b1_ref[i]
            xs = jnp.dot(h, ws_ref[i], preferred_element_type=f32)
            pre = xs[:, None, :] + xr[None, :, :]
            a1 = jnp.tanh(pre).reshape(V * Vp, H)
            a2 = jnp.tanh(jnp.dot(a1, f2_ref[i], preferred_element_type=f32)
                          + b2_ref[i])
            agg = agg + jnp.dot(aggws[i], a2, preferred_element_type=f32)
        inp_r = jnp.dot(ins, irw_ref[...], preferred_element_type=f32) + irb_ref[...]
        inp_i = jnp.dot(ins, iiw_ref[...], preferred_element_type=f32) + iib_ref[...]
        inp_n = jnp.dot(ins, inw_ref[...], preferred_element_type=f32) + inb_ref[...]
        r = jax.nn.sigmoid(inp_r + jnp.dot(agg, hrw_ref[...], preferred_element_type=f32))
        ig = jax.nn.sigmoid(inp_i + jnp.dot(agg, hiw_ref[...], preferred_element_type=f32))
        n = jnp.tanh(inp_n + r * jnp.dot(agg, hhw_ref[...], preferred_element_type=f32))
        h = (1.0 - ig) * n + ig * h
        p = jax.nn.relu(jnp.dot(h, o1w_ref[...], preferred_element_type=f32) + o1b_ref[...])
        p = jax.nn.relu(jnp.dot(p, o2w_ref[...], preferred_element_type=f32) + o2b_ref[...])
        p = jnp.dot(p, o3w_ref[...], preferred_element_type=f32) + o3b_ref[...]
        ins = ins + p
        out_ref[0, t] = ins


def kernel(inputs, sampled_edges, msg_fc1_w, msg_fc1_b, msg_fc2_w, msg_fc2_b,
           hidden_r_w, hidden_i_w, hidden_h_w, input_r_w, input_r_b,
           input_i_w, input_i_b, input_n_w, input_n_b, out_fc1_w, out_fc1_b,
           out_fc2_w, out_fc2_b, out_fc3_w, out_fc3_b):
    B, T, V, DIN = inputs.shape
    H = hidden_r_w.shape[0]
    ET = msg_fc1_w.shape[0]
    NT = ET - 1
    Vp = ((V + 7) // 8) * 8

    vals = jnp.transpose(sampled_edges[:, :, 1:], (0, 2, 1))
    vals = vals.reshape(B, NT, V, V - 1)
    zcol = jnp.zeros((B, NT, V, 1), dtype=jnp.float32)
    left = jnp.concatenate([vals, zcol], axis=-1)
    right = jnp.concatenate([zcol, vals], axis=-1)
    rr = jnp.arange(V)[None, :]
    ss = jnp.arange(V)[:, None]
    w = jnp.where(rr < ss, left, 0.0) + jnp.where(rr > ss, right, 0.0)
    wt = jnp.swapaxes(w, -1, -2)

    scale = 1.0 / ((ET - 1.0) * (V - 1.0))
    rmat = np.zeros((V, V * Vp), dtype=np.float32)
    rmat[np.arange(V)[:, None], np.arange(V)[:, None] * Vp +
         np.arange(Vp)[None, :]] = 1.0
    cols = np.arange(V * Vp)
    mask = ((cols % Vp)[None, :] == np.arange(V)[:, None]).astype(np.float32)
    mask = mask * scale
    rmat = jnp.asarray(rmat)
    mask = jnp.asarray(mask)

    wr = msg_fc1_w[1:, :H, :]
    ws = msg_fc1_w[1:, H:, :]
    b1 = msg_fc1_b[1:].reshape(NT, 1, H)
    f2 = msg_fc2_w[1:]
    b2 = msg_fc2_b[1:].reshape(NT, 1, H)
    ins0 = inputs[:, 0]

    def vec(x):
        return x.reshape(1, -1)

    def rep(shape):
        nd = len(shape)
        return pl.BlockSpec(shape, lambda b, _n=nd: (0,) * _n)

    import functools
    body = functools.partial(_decoder_kernel, T, V, Vp, NT)

    return pl.pallas_call(
        body,
        grid=(B,),
        in_specs=[
            pl.BlockSpec((1, V, DIN), lambda b: (b, 0, 0)),
            pl.BlockSpec((1, NT, V, V), lambda b: (b, 0, 0, 0)),
            rep((V, V * Vp)), rep((V, V * Vp)),
            rep((NT, H, H)), rep((NT, H, H)), rep((NT, 1, H)),
            rep((NT, H, H)), rep((NT, 1, H)),
            rep((H, H)), rep((H, H)), rep((H, H)),
            rep((DIN, H)), rep((1, H)),
            rep((DIN, H)), rep((1, H)),
            rep((DIN, H)), rep((1, H)),
            rep((H, H)), rep((1, H)),
            rep((H, H)), rep((1, H)),
            rep((H, DIN)), rep((1, DIN)),
        ],
        out_specs=pl.BlockSpec((1, T, V, DIN), lambda b: (b, 0, 0, 0)),
        out_shape=jax.ShapeDtypeStruct((B, T, V, DIN), jnp.float32),
        compiler_params=pltpu.CompilerParams(
            dimension_semantics=("parallel",)),
    )(ins0, wt, mask, rmat,
      wr, ws, b1, f2, b2,
      hidden_r_w, hidden_i_w, hidden_h_w,
      input_r_w, vec(input_r_b), input_i_w, vec(input_i_b),
      input_n_w, vec(input_n_b),
      out_fc1_w, vec(out_fc1_b), out_fc2_w, vec(out_fc2_b),
      out_fc3_w, vec(out_fc3_b))
